# Initial kernel scaffold; baseline (speedup 1.0000x reference)
import jax
import jax.numpy as jnp
from jax.experimental import pallas as pl


def kernel(features, scaled_improvement_down, scaled_improvement_up, variable_chosen, h, c, iou, branch_cands, gains, W_iou, U_iou, b_iou, W_f_W, W_f_b, b_f, U_f_W, U_f_b, lin_W, lin_b):
    raise NotImplementedError("write your pallas kernel here")


# speedup vs baseline: 15.5843x; 15.5843x over previous
import jax
import jax.numpy as jnp
from jax.experimental import pallas as pl
from jax.experimental.pallas import tpu as pltpu

N = 10000
B = 400
NCHUNK = N // B
X = 128
H = 128
D3 = 3 * H
D = D3 + H
C = 64
NP = 10240


def _mm_kernel(x_ref, w_ref, b_ref, o_ref):
    o_ref[...] = (
        jnp.dot(x_ref[...], w_ref[...], preferred_element_type=jnp.float32)
        + b_ref[...]
    )


def _scan_kernel(p_ref, u_ref, sid_ref, siu_ref, linw_ref, linb_ref,
                 down_ref, up_ref,
                 h_s, c_s, iou0_s, hrows_s):
    p = pl.program_id(0)
    j = pl.program_id(1)

    @pl.when((p == 0) & (j == 0))
    def _init():
        h_s[...] = jnp.zeros_like(h_s)
        c_s[...] = jnp.zeros_like(c_s)

    h0 = h_s[...]
    c0 = c_s[...]
    U = u_ref[...]

    @pl.when(p == 0)
    def _pass1():
        def body(t, carry):
            h_prev, c_prev, _ = carry
            i = B - 1 - t
            prow = p_ref[pl.ds(i, 1), :]
            z = jnp.dot(h_prev, U, preferred_element_type=jnp.float32)
            f = jax.nn.sigmoid(prow[:, D3:D] + z[:, D3:D])
            c_red = f * c_prev
            iou_a = prow[:, 0:D3] + z[:, 0:D3]
            ig = jax.nn.sigmoid(iou_a[:, 0:H])
            og = jax.nn.sigmoid(iou_a[:, H:2 * H])
            ug = jnp.tanh(iou_a[:, 2 * H:D3])
            c_new = ig * ug + c_red
            h_new = og * jnp.tanh(c_new)
            return h_new, c_new, z[:, 0:D3]

        h_f, c_f, ziou = jax.lax.fori_loop(
            0, B, body, (h0, c0, jnp.zeros((1, D3), jnp.float32)))
        h_s[...] = h_f
        c_s[...] = c_f
        iou0_s[...] = ziou

    @pl.when(p == 1)
    def _pass2():
        iou0_v = iou0_s[...]

        def body(i, carry):
            h_prev, c_prev = carry
            prow = p_ref[pl.ds(i, 1), :]
            z = jnp.dot(h_prev, U, preferred_element_type=jnp.float32)
            first2 = (j == 0) & (i == 0)
            ziou = jnp.where(first2, iou0_v, z[:, 0:D3])
            f = jax.nn.sigmoid(prow[:, D3:D] + z[:, D3:D])
            c_base = jnp.where(first2, c_prev, f * c_prev)
            iou_a = prow[:, 0:D3] + ziou
            ig = jax.nn.sigmoid(iou_a[:, 0:H])
            og = jax.nn.sigmoid(iou_a[:, H:2 * H])
            ug = jnp.tanh(iou_a[:, 2 * H:D3])
            c_new = ig * ug + c_base
            h_new = og * jnp.tanh(c_new)
            hrows_s[pl.ds(i, 1), :] = h_new
            return h_new, c_new

        h_f, c_f = jax.lax.fori_loop(0, B, body, (h0, c0))
        h_s[...] = h_f
        c_s[...] = c_f
        hsc = (jnp.sum(hrows_s[...] * linw_ref[...], axis=1, keepdims=True)
               + linb_ref[...])
        down_ref[...] = hsc * sid_ref[...]
        up_ref[...] = hsc * siu_ref[...]


def _cand_kernel(v_ref, d_ref, u_ref, cand_ref, g0_ref, g1_ref,
                 scores_ref, best_ref):
    cands = cand_ref[...]
    CH = 1024

    def body(k, acc):
        pd_a, pu_a, cnt_a = acc
        v = v_ref[pl.ds(k * CH, CH), :]
        d = d_ref[pl.ds(k * CH, CH), :]
        u = u_ref[pl.ds(k * CH, CH), :]
        mask = (v == cands).astype(jnp.float32)
        pd_a = pd_a + jnp.sum(mask * d, axis=0, keepdims=True)
        pu_a = pu_a + jnp.sum(mask * u, axis=0, keepdims=True)
        cnt_a = cnt_a + jnp.sum(mask, axis=0, keepdims=True)
        return pd_a, pu_a, cnt_a

    zero = jnp.zeros((1, C), jnp.float32)
    pd, pu, cnt = jax.lax.fori_loop(0, NP // CH, body, (zero, zero, zero))
    denom = jnp.where(cnt > 0, cnt, 1.0)
    pd = pd / denom
    pu = pu / denom
    has = cnt > 0
    pd_f = jnp.where(has, pd, g0_ref[...])
    pu_f = jnp.where(has, pu, g1_ref[...])
    scores = 0.5 * pd_f + 0.5 * jnp.maximum(pd_f, pu_f)
    scores_ref[...] = scores
    best_ref[...] = jnp.argmax(scores, axis=1).astype(jnp.int32).reshape(1, 1)


@jax.jit
def kernel(features, scaled_improvement_down, scaled_improvement_up,
           variable_chosen, h, c, iou, branch_cands, gains,
           W_iou, U_iou, b_iou, W_f_W, W_f_b, b_f, U_f_W, U_f_b,
           lin_W, lin_b):
    f32 = jnp.float32

    Wcat = jnp.concatenate([W_iou.T, W_f_W.T], axis=1)
    bias = jnp.concatenate([b_iou[0], W_f_b + U_f_b + b_f[0]])
    bias = bias.reshape(1, D)
    P = pl.pallas_call(
        _mm_kernel,
        grid=(NCHUNK,),
        in_specs=[
            pl.BlockSpec((B, X), lambda i: (i, 0)),
            pl.BlockSpec((X, D), lambda i: (0, 0)),
            pl.BlockSpec((1, D), lambda i: (0, 0)),
        ],
        out_specs=pl.BlockSpec((B, D), lambda i: (i, 0)),
        out_shape=jax.ShapeDtypeStruct((N, D), f32),
    )(features, Wcat, bias)

    Ucat = jnp.concatenate([U_iou.T, U_f_W.T], axis=1)
    sid = scaled_improvement_down.reshape(N, 1)
    siu = scaled_improvement_up.reshape(N, 1)
    linw = lin_W.reshape(1, H)
    linb = lin_b.reshape(1, 1)

    down, up = pl.pallas_call(
        _scan_kernel,
        grid=(2, NCHUNK),
        in_specs=[
            pl.BlockSpec((B, D), lambda p, j: (jnp.where(p == 0, NCHUNK - 1 - j, j), 0)),
            pl.BlockSpec((H, D), lambda p, j: (0, 0)),
            pl.BlockSpec((B, 1), lambda p, j: (jnp.where(p == 0, NCHUNK - 1 - j, j), 0)),
            pl.BlockSpec((B, 1), lambda p, j: (jnp.where(p == 0, NCHUNK - 1 - j, j), 0)),
            pl.BlockSpec((1, H), lambda p, j: (0, 0)),
            pl.BlockSpec((1, 1), lambda p, j: (0, 0)),
        ],
        out_specs=[
            pl.BlockSpec((B, 1), lambda p, j: (jnp.where(p == 0, NCHUNK - 1 - j, j), 0)),
            pl.BlockSpec((B, 1), lambda p, j: (jnp.where(p == 0, NCHUNK - 1 - j, j), 0)),
        ],
        out_shape=[
            jax.ShapeDtypeStruct((N, 1), f32),
            jax.ShapeDtypeStruct((N, 1), f32),
        ],
        scratch_shapes=[
            pltpu.VMEM((1, H), jnp.float32),
            pltpu.VMEM((1, H), jnp.float32),
            pltpu.VMEM((1, D3), jnp.float32),
            pltpu.VMEM((B, H), jnp.float32),
        ],
    )(P, Ucat, sid, siu, linw, linb)

    down_scores = down[:, 0]
    up_scores = up[:, 0]

    pad = NP - N
    vars_f = jnp.pad(variable_chosen.astype(f32), (0, pad),
                     constant_values=-1.0).reshape(NP, 1)
    d_p = jnp.pad(down_scores, (0, pad)).reshape(NP, 1)
    u_p = jnp.pad(up_scores, (0, pad)).reshape(NP, 1)
    cands_row = branch_cands.astype(f32).reshape(1, C)
    g0 = gains[:, 0].reshape(1, C)
    g1 = gains[:, 1].reshape(1, C)

    scores2, best2 = pl.pallas_call(
        _cand_kernel,
        grid=(1,),
        in_specs=[
            pl.BlockSpec((NP, 1), lambda i: (0, 0)),
            pl.BlockSpec((NP, 1), lambda i: (0, 0)),
            pl.BlockSpec((NP, 1), lambda i: (0, 0)),
            pl.BlockSpec((1, C), lambda i: (0, 0)),
            pl.BlockSpec((1, C), lambda i: (0, 0)),
            pl.BlockSpec((1, C), lambda i: (0, 0)),
        ],
        out_specs=[
            pl.BlockSpec((1, C), lambda i: (0, 0)),
            pl.BlockSpec((1, 1), lambda i: (0, 0)),
        ],
        out_shape=[
            jax.ShapeDtypeStruct((1, C), f32),
            jax.ShapeDtypeStruct((1, 1), jnp.int32),
        ],
    )(vars_f, d_p, u_p, cands_row, g0, g1)

    scores = scores2[0]
    best_var = best2[0, 0]
    return (best_var, scores, down_scores, up_scores, variable_chosen)
